# baseline (speedup 1.0000x reference)
import functools

import jax
import jax.numpy as jnp
from jax import lax
from jax.experimental import pallas as pl
from jax.experimental.pallas import tpu as pltpu
from jax.experimental.pallas import tpu_sc as plsc

_B = 16384
_IN = 64
_HID = 128
_EMB = 32
_PACK = 128 // _EMB
_ROWS = 1000000 // _PACK
_NC = 2
_NS = 16
_NW = _NC * _NS
_BPW = _B // _NW
_CHUNK = 128
_NCHUNK = _BPW // _CHUNK


def _sc_gather_body(table_hbm, idx_hbm, out_hbm, idx_v, rows_v, sem):
    wid = lax.axis_index("s") * _NC + lax.axis_index("c")
    base = wid * _BPW
    pltpu.sync_copy(idx_hbm.at[wid], idx_v)
    copies = []
    for j in range(_NCHUNK):
        copies.append(
            pltpu.async_copy(
                table_hbm.at[idx_v.at[j]],
                rows_v.at[pl.ds(j * _CHUNK, _CHUNK)],
                sem,
            )
        )
    for c in copies:
        c.wait()
    pltpu.sync_copy(rows_v, out_hbm.at[pl.ds(base, _BPW)])


_sc_gather = functools.partial(
    pl.kernel,
    out_type=jax.ShapeDtypeStruct((_B, 128), jnp.float32),
    mesh=plsc.VectorSubcoreMesh(core_axis_name="c", subcore_axis_name="s"),
    scratch_types=[
        pltpu.VMEM((_NCHUNK, _CHUNK), jnp.int32),
        pltpu.VMEM((_BPW, 128), jnp.float32),
        pltpu.SemaphoreType.DMA,
    ],
)(_sc_gather_body)


def _mlp_body(x_ref, r_ref, m_ref, w1x_ref, w1e_ref, b1_ref, w2_ref, b2_ref, o_ref):
    blk = x_ref.shape[0]
    group = lax.broadcasted_iota(jnp.int32, (blk, 128), 1) // _EMB
    e = jnp.where(group == m_ref[...], r_ref[...], 0.0)
    h = jnp.dot(x_ref[...], w1x_ref[...], preferred_element_type=jnp.float32)
    h = h + jnp.dot(e, w1e_ref[...], preferred_element_type=jnp.float32)
    h = jnp.maximum(h + b1_ref[...], 0.0)
    o_ref[...] = jnp.sum(h * w2_ref[...], axis=1, keepdims=True) + b2_ref[0, 0]


def kernel(x, user_ids, emb_table, W1, b1, W2, b2):
    ids = user_ids.astype(jnp.int32)
    idq = (ids // _PACK).reshape(_NW, _NCHUNK, _CHUNK)
    idr = (ids % _PACK).reshape(_B, 1)
    table = emb_table.reshape(_ROWS, 128)
    rows = _sc_gather(table, idq)

    w1t = W1.T
    w1x = w1t[:_IN]
    w1e4 = jnp.concatenate([w1t[_IN:]] * _PACK, axis=0)

    blk = 2048
    out = pl.pallas_call(
        _mlp_body,
        grid=(_B // blk,),
        in_specs=[
            pl.BlockSpec((blk, _IN), lambda i: (i, 0)),
            pl.BlockSpec((blk, 128), lambda i: (i, 0)),
            pl.BlockSpec((blk, 1), lambda i: (i, 0)),
            pl.BlockSpec((_IN, _HID), lambda i: (0, 0)),
            pl.BlockSpec((128, _HID), lambda i: (0, 0)),
            pl.BlockSpec((1, _HID), lambda i: (0, 0)),
            pl.BlockSpec((1, _HID), lambda i: (0, 0)),
            pl.BlockSpec(memory_space=pltpu.SMEM),
        ],
        out_specs=pl.BlockSpec((blk, 1), lambda i: (i, 0)),
        out_shape=jax.ShapeDtypeStruct((_B, 1), jnp.float32),
    )(x, rows, idr, w1x, w1e4, b1.reshape(1, _HID), W2, b2.reshape(1, 1))
    return out


# speedup vs baseline: 3.4421x; 3.4421x over previous
import functools

import jax
import jax.numpy as jnp
from jax import lax
from jax.experimental import pallas as pl
from jax.experimental.pallas import tpu as pltpu
from jax.experimental.pallas import tpu_sc as plsc

_B = 16384
_IN = 64
_HID = 128
_EMB = 32
_NU = 1000000
_NC = 2
_NS = 16
_NW = _NC * _NS
_BPW = _B // _NW
_BS = 16
_NBATCH = _BPW // _BS


def _sc_gather_body(table_hbm, idx_hbm, out_hbm, idx_v, staged, cols, sem):
    wid = lax.axis_index("s") * _NC + lax.axis_index("c")
    base = wid * _BPW
    pltpu.sync_copy(idx_hbm.at[wid], idx_v)

    j16 = lax.iota(jnp.int32, 16)
    j16b = j16 + 16

    def batch(g, carry):
        uids = plsc.load_gather(idx_v, [g * _BS + j16])
        copies = []
        for m in range(_BS):
            uid = uids[m]
            cb = pl.multiple_of(uid - uid % 128, 128)
            copies.append(
                pltpu.async_copy(
                    table_hbm.at[:, pl.ds(cb, 128)],
                    staged.at[:, pl.ds(m * 128, 128)],
                    sem,
                )
            )
        for c in copies:
            c.wait()
        for m in range(_BS):
            k = g * _BS + m
            uid = uids[m]
            col = jnp.full((16,), m * 128 + uid % 128, jnp.int32)
            dst = jnp.full((16,), k, jnp.int32)
            v0 = plsc.load_gather(staged, [j16, col])
            v1 = plsc.load_gather(staged, [j16b, col])
            plsc.store_scatter(cols, [j16, dst], v0)
            plsc.store_scatter(cols, [j16b, dst], v1)
        return carry

    lax.fori_loop(0, _NBATCH, batch, 0)
    pltpu.sync_copy(cols, out_hbm.at[:, pl.ds(base, _BPW)])


_sc_gather = functools.partial(
    pl.kernel,
    out_type=jax.ShapeDtypeStruct((_EMB, _B), jnp.float32),
    mesh=plsc.VectorSubcoreMesh(core_axis_name="c", subcore_axis_name="s"),
    scratch_types=[
        pltpu.VMEM((_BPW,), jnp.int32),
        pltpu.VMEM((_EMB, _BS * 128), jnp.float32),
        pltpu.VMEM((_EMB, _BPW), jnp.float32),
        pltpu.SemaphoreType.DMA,
    ],
    compiler_params=pltpu.CompilerParams(needs_layout_passes=False),
)(_sc_gather_body)


def _mlp_body(xt_ref, et_ref, w1x_ref, w1e_ref, b1_ref, w2_ref, b2_ref, o_ref):
    dn = (((0,), (0,)), ((), ()))
    h = lax.dot_general(xt_ref[...], w1x_ref[...], dn,
                        preferred_element_type=jnp.float32)
    h = h + lax.dot_general(et_ref[...], w1e_ref[...], dn,
                            preferred_element_type=jnp.float32)
    h = jnp.maximum(h + b1_ref[...], 0.0)
    dn2 = (((1,), (1,)), ((), ()))
    o_ref[...] = lax.dot_general(w2_ref[...], h, dn2,
                                 preferred_element_type=jnp.float32) + b2_ref[0, 0]


def kernel(x, user_ids, emb_table, W1, b1, W2, b2):
    ids = user_ids.astype(jnp.int32).reshape(_NW, _BPW)
    table_t = emb_table.T
    et = _sc_gather(table_t, ids)

    xt = x.T
    w1t = W1.T
    w1x = w1t[:_IN]
    w1e = w1t[_IN:]

    blk = 2048
    out_t = pl.pallas_call(
        _mlp_body,
        grid=(_B // blk,),
        in_specs=[
            pl.BlockSpec((_IN, blk), lambda i: (0, i)),
            pl.BlockSpec((_EMB, blk), lambda i: (0, i)),
            pl.BlockSpec((_IN, _HID), lambda i: (0, 0)),
            pl.BlockSpec((_EMB, _HID), lambda i: (0, 0)),
            pl.BlockSpec((1, _HID), lambda i: (0, 0)),
            pl.BlockSpec((1, _HID), lambda i: (0, 0)),
            pl.BlockSpec(memory_space=pltpu.SMEM),
        ],
        out_specs=pl.BlockSpec((1, blk), lambda i: (0, i)),
        out_shape=jax.ShapeDtypeStruct((1, _B), jnp.float32),
    )(xt, et, w1x, w1e, b1.reshape(1, _HID), W2, b2.reshape(1, 1))
    return out_t.T


# speedup vs baseline: 3.6413x; 1.0579x over previous
import functools

import jax
import jax.numpy as jnp
from jax import lax
from jax.experimental import pallas as pl
from jax.experimental.pallas import tpu as pltpu
from jax.experimental.pallas import tpu_sc as plsc

_B = 16384
_IN = 64
_HID = 128
_EMB = 32
_NU = 1000000
_NC = 2
_NS = 16
_NW = _NC * _NS
_BPW = _B // _NW
_BS = 8
_NBATCH = _BPW // _BS


def _sc_gather_body(table_hbm, idx_hbm, out_hbm, idx_v, st0, st1, cols, sem0, sem1):
    wid = lax.axis_index("s") * _NC + lax.axis_index("c")
    base = wid * _BPW
    pltpu.sync_copy(idx_hbm.at[wid], idx_v.at[pl.ds(0, _BPW)])

    j16 = lax.iota(jnp.int32, 16)
    j16b = j16 + 16

    def fire(g, st, sem):
        uids = plsc.load_gather(idx_v, [g * _BS + j16])
        for m in range(_BS):
            uid = uids[m]
            cb = pl.multiple_of(uid - uid % 128, 128)
            pltpu.async_copy(
                table_hbm.at[:, pl.ds(cb, 128)],
                st.at[:, pl.ds(m * 128, 128)],
                sem,
            )

    def drain(st, sem):
        for m in range(_BS):
            pltpu.make_async_copy(
                table_hbm.at[:, pl.ds(0, 128)],
                st.at[:, pl.ds(m * 128, 128)],
                sem,
            ).wait()

    def extract(g, st):
        uids = plsc.load_gather(idx_v, [g * _BS + j16])
        for m in range(_BS):
            k = g * _BS + m
            uid = uids[m]
            col = jnp.full((16,), m * 128 + uid % 128, jnp.int32)
            dst = jnp.full((16,), k, jnp.int32)
            v0 = plsc.load_gather(st, [j16, col])
            v1 = plsc.load_gather(st, [j16b, col])
            plsc.store_scatter(cols, [j16, dst], v0)
            plsc.store_scatter(cols, [j16b, dst], v1)

    fire(0, st0, sem0)

    def pair(h, carry):
        g0 = 2 * h
        fire(g0 + 1, st1, sem1)
        drain(st0, sem0)
        extract(g0, st0)

        @pl.when(g0 + 2 < _NBATCH)
        def _():
            fire(g0 + 2, st0, sem0)

        drain(st1, sem1)
        extract(g0 + 1, st1)
        return carry

    lax.fori_loop(0, _NBATCH // 2, pair, 0)
    pltpu.sync_copy(cols, out_hbm.at[:, pl.ds(base, _BPW)])


_sc_gather = functools.partial(
    pl.kernel,
    out_type=jax.ShapeDtypeStruct((_EMB, _B), jnp.float32),
    mesh=plsc.VectorSubcoreMesh(core_axis_name="c", subcore_axis_name="s"),
    scratch_types=[
        pltpu.VMEM((_BPW + 16,), jnp.int32),
        pltpu.VMEM((_EMB, _BS * 128), jnp.float32),
        pltpu.VMEM((_EMB, _BS * 128), jnp.float32),
        pltpu.VMEM((_EMB, _BPW), jnp.float32),
        pltpu.SemaphoreType.DMA,
        pltpu.SemaphoreType.DMA,
    ],
    compiler_params=pltpu.CompilerParams(needs_layout_passes=False),
)(_sc_gather_body)


def _mlp_body(xt_ref, et_ref, w1x_ref, w1e_ref, b1_ref, w2_ref, b2_ref, o_ref):
    dn = (((0,), (0,)), ((), ()))
    h = lax.dot_general(xt_ref[...], w1x_ref[...], dn,
                        preferred_element_type=jnp.float32)
    h = h + lax.dot_general(et_ref[...], w1e_ref[...], dn,
                            preferred_element_type=jnp.float32)
    h = jnp.maximum(h + b1_ref[...], 0.0)
    dn2 = (((1,), (1,)), ((), ()))
    o_ref[...] = lax.dot_general(w2_ref[...], h, dn2,
                                 preferred_element_type=jnp.float32) + b2_ref[0, 0]


def kernel(x, user_ids, emb_table, W1, b1, W2, b2):
    ids = user_ids.astype(jnp.int32).reshape(_NW, _BPW)
    table_t = emb_table.T
    et = _sc_gather(table_t, ids)

    xt = x.T
    w1t = W1.T
    w1x = w1t[:_IN]
    w1e = w1t[_IN:]

    blk = 2048
    out_t = pl.pallas_call(
        _mlp_body,
        grid=(_B // blk,),
        in_specs=[
            pl.BlockSpec((_IN, blk), lambda i: (0, i)),
            pl.BlockSpec((_EMB, blk), lambda i: (0, i)),
            pl.BlockSpec((_IN, _HID), lambda i: (0, 0)),
            pl.BlockSpec((_EMB, _HID), lambda i: (0, 0)),
            pl.BlockSpec((1, _HID), lambda i: (0, 0)),
            pl.BlockSpec((1, _HID), lambda i: (0, 0)),
            pl.BlockSpec(memory_space=pltpu.SMEM),
        ],
        out_specs=pl.BlockSpec((1, blk), lambda i: (0, i)),
        out_shape=jax.ShapeDtypeStruct((1, _B), jnp.float32),
    )(xt, et, w1x, w1e, b1.reshape(1, _HID), W2, b2.reshape(1, 1))
    return out_t.T


# speedup vs baseline: 4.0352x; 1.1082x over previous
import functools

import jax
import jax.numpy as jnp
from jax import lax
from jax.experimental import pallas as pl
from jax.experimental.pallas import tpu as pltpu
from jax.experimental.pallas import tpu_sc as plsc

_B = 16384
_IN = 64
_HID = 128
_EMB = 32
_NU = 1000000
_NC = 2
_NS = 16
_NW = _NC * _NS
_BPW = _B // _NW
_BS = 4
_NBATCH = _BPW // _BS
_NBUF = 4


def _sc_gather_body(table_hbm, idx_hbm, out_hbm, idx_v, st0, st1, st2, st3,
                    cols, sem0, sem1, sem2, sem3):
    wid = lax.axis_index("s") * _NC + lax.axis_index("c")
    base = wid * _BPW
    pltpu.sync_copy(idx_hbm.at[wid], idx_v.at[pl.ds(0, _BPW)])

    sts = (st0, st1, st2, st3)
    sems = (sem0, sem1, sem2, sem3)
    j16 = lax.iota(jnp.int32, 16)
    j16b = j16 + 16

    def fire(g, st, sem):
        uids = plsc.load_gather(idx_v, [g * _BS + j16])
        for m in range(_BS):
            uid = uids[m]
            cb = pl.multiple_of(uid - uid % 128, 128)
            pltpu.async_copy(
                table_hbm.at[:, pl.ds(cb, 128)],
                st.at[:, pl.ds(m * 128, 128)],
                sem,
            )

    def drain(st, sem):
        for m in range(_BS):
            pltpu.make_async_copy(
                table_hbm.at[:, pl.ds(0, 128)],
                st.at[:, pl.ds(m * 128, 128)],
                sem,
            ).wait()

    def extract(g, st):
        uids = plsc.load_gather(idx_v, [g * _BS + j16])
        for m in range(_BS):
            k = g * _BS + m
            uid = uids[m]
            col = jnp.full((16,), m * 128 + uid % 128, jnp.int32)
            dst = jnp.full((16,), k, jnp.int32)
            v0 = plsc.load_gather(st, [j16, col])
            v1 = plsc.load_gather(st, [j16b, col])
            plsc.store_scatter(cols, [j16, dst], v0)
            plsc.store_scatter(cols, [j16b, dst], v1)

    for r in range(_NBUF - 1):
        fire(r, sts[r], sems[r])

    def quad(h, carry):
        for r in range(_NBUF):
            g = _NBUF * h + r
            rn = (r + _NBUF - 1) % _NBUF

            @pl.when(g + _NBUF - 1 < _NBATCH)
            def _():
                fire(g + _NBUF - 1, sts[rn], sems[rn])

            drain(sts[r], sems[r])
            extract(g, sts[r])
        return carry

    lax.fori_loop(0, _NBATCH // _NBUF, quad, 0)
    pltpu.sync_copy(cols, out_hbm.at[:, pl.ds(base, _BPW)])


_sc_gather = functools.partial(
    pl.kernel,
    out_type=jax.ShapeDtypeStruct((_EMB, _B), jnp.float32),
    mesh=plsc.VectorSubcoreMesh(core_axis_name="c", subcore_axis_name="s"),
    scratch_types=[
        pltpu.VMEM((_BPW + 16,), jnp.int32),
        pltpu.VMEM((_EMB, _BS * 128), jnp.float32),
        pltpu.VMEM((_EMB, _BS * 128), jnp.float32),
        pltpu.VMEM((_EMB, _BS * 128), jnp.float32),
        pltpu.VMEM((_EMB, _BS * 128), jnp.float32),
        pltpu.VMEM((_EMB, _BPW), jnp.float32),
        pltpu.SemaphoreType.DMA,
        pltpu.SemaphoreType.DMA,
        pltpu.SemaphoreType.DMA,
        pltpu.SemaphoreType.DMA,
    ],
    compiler_params=pltpu.CompilerParams(needs_layout_passes=False),
)(_sc_gather_body)


def _mlp_body(xt_ref, et_ref, w1x_ref, w1e_ref, b1_ref, w2_ref, b2_ref, o_ref):
    dn = (((0,), (0,)), ((), ()))
    h = lax.dot_general(xt_ref[...], w1x_ref[...], dn,
                        preferred_element_type=jnp.float32)
    h = h + lax.dot_general(et_ref[...], w1e_ref[...], dn,
                            preferred_element_type=jnp.float32)
    h = jnp.maximum(h + b1_ref[...], 0.0)
    dn2 = (((1,), (1,)), ((), ()))
    o_ref[...] = lax.dot_general(w2_ref[...], h, dn2,
                                 preferred_element_type=jnp.float32) + b2_ref[0, 0]


def kernel(x, user_ids, emb_table, W1, b1, W2, b2):
    ids = user_ids.astype(jnp.int32).reshape(_NW, _BPW)
    table_t = emb_table.T
    et = _sc_gather(table_t, ids)

    xt = x.T
    w1t = W1.T
    w1x = w1t[:_IN]
    w1e = w1t[_IN:]

    blk = 2048
    out_t = pl.pallas_call(
        _mlp_body,
        grid=(_B // blk,),
        in_specs=[
            pl.BlockSpec((_IN, blk), lambda i: (0, i)),
            pl.BlockSpec((_EMB, blk), lambda i: (0, i)),
            pl.BlockSpec((_IN, _HID), lambda i: (0, 0)),
            pl.BlockSpec((_EMB, _HID), lambda i: (0, 0)),
            pl.BlockSpec((1, _HID), lambda i: (0, 0)),
            pl.BlockSpec((1, _HID), lambda i: (0, 0)),
            pl.BlockSpec(memory_space=pltpu.SMEM),
        ],
        out_specs=pl.BlockSpec((1, blk), lambda i: (0, i)),
        out_shape=jax.ShapeDtypeStruct((1, _B), jnp.float32),
    )(xt, et, w1x, w1e, b1.reshape(1, _HID), W2, b2.reshape(1, 1))
    return out_t.T
